# baseline (speedup 1.0000x reference)
import functools

import jax
import jax.numpy as jnp
import numpy as np
from jax import lax
from jax.experimental import pallas as pl
from jax.experimental.pallas import tpu as pltpu
from jax.experimental.pallas import tpu_sc as plsc

NC = 2
NS = 16
NW = NC * NS
G = 4


def _ffm_body(m, k, spt, offs_np, x_hbm, tab_hbm, out_hbm,
              xv, buf, offs_s, outv, sem0, sem1):
    npairs = offs_np.shape[1]
    ng = spt // G
    c = lax.axis_index("c")
    s = lax.axis_index("s")
    wid = s * NC + c
    base = wid * spt

    pltpu.sync_copy(x_hbm.at[pl.ds(base * m, spt * m)], xv)
    for q in range(4):
        for p in range(offs_np.shape[1]):
            offs_s[q, p] = jnp.int32(int(offs_np[q, p]))

    def issue(g, b):
        idx = xv.at[pl.ds(g * (G * m), G * m)]
        sem = sem0 if b == 0 else sem1
        pltpu.async_copy(tab_hbm.at[idx], buf.at[b], sem)

    def wait_grp(b):
        idx = xv.at[pl.ds(0, G * m)]
        sem = sem0 if b == 0 else sem1
        pltpu.make_async_copy(tab_hbm.at[idx], buf.at[b], sem).wait()

    issue(0, 0)

    lane = lax.iota(jnp.int32, 16)

    def sample_body(par, g, t, zvec):
        tm = t * m
        zero = jnp.zeros((k,), jnp.float32)
        accs = (zero, zero, zero, zero, zero, zero, zero, zero)

        def pbody(p, acc):
            r1 = offs_s[0, p]
            c1 = offs_s[1, p]
            r2 = offs_s[2, p]
            c2 = offs_s[3, p]
            v1 = buf[par, tm + r1, pl.ds(c1, k)]
            v2 = buf[par, tm + r2, pl.ds(c2, k)]
            return acc[1:] + (acc[0] + v1 * v2,)

        accs = plsc.parallel_loop(0, 325, 1, unroll=5, carry=accs)(pbody)
        tot = (((accs[0] + accs[1]) + (accs[2] + accs[3]))
               + ((accs[4] + accs[5]) + (accs[6] + accs[7])))
        z = lax.reduce_sum(tot, axes=(0,))
        ls = g * G + t
        zvec = jnp.where(lane == lax.rem(ls, 16), z, zvec)

        @pl.when(lax.rem(ls, 16) == 15)
        def _():
            chunk = (ls // 16) * 16
            outv[pl.ds(chunk, 16)] = 1.0 / (1.0 + jnp.exp(-zvec))

        return zvec

    def group_body(g, zvec):
        par = lax.rem(g, 2)
        more = g + 1 < ng

        @pl.when(jnp.logical_and(more, par == 0))
        def _():
            issue(g + 1, 1)

        @pl.when(jnp.logical_and(more, par == 1))
        def _():
            issue(g + 1, 0)

        @pl.when(par == 0)
        def _():
            wait_grp(0)

        @pl.when(par == 1)
        def _():
            wait_grp(1)

        return lax.fori_loop(
            0, G, lambda t, zv: sample_body(par, g, t, zv), zvec)

    lax.fori_loop(0, ng, group_body, jnp.zeros((16,), jnp.float32))

    pltpu.sync_copy(outv, out_hbm.at[pl.ds(base, spt)])


def kernel(x, field_embeddings):
    n, m, k = field_embeddings.shape
    if x.ndim == 1:
        x = x[None, :]
    b = x.shape[0]
    x = (x.astype(jnp.int32) % n).astype(jnp.int32)

    bpad = ((b + NW * 16 - 1) // (NW * 16)) * (NW * 16)
    if bpad != b:
        x = jnp.concatenate(
            [x, jnp.zeros((bpad - b, m), jnp.int32)], axis=0)
    spt = bpad // NW

    tab = field_embeddings.reshape(n, m * k)
    xflat = x.reshape(-1)
    i1, i2 = np.triu_indices(m, k=1)
    offs_np = np.stack([i1, i2 * k, i2, i1 * k]).astype(np.int32)

    mesh = plsc.VectorSubcoreMesh(
        core_axis_name="c", subcore_axis_name="s",
        num_cores=NC, num_subcores=NS)
    fn = pl.kernel(
        functools.partial(_ffm_body, m, k, spt, offs_np),
        out_type=jax.ShapeDtypeStruct((bpad,), jnp.float32),
        mesh=mesh,
        compiler_params=pltpu.CompilerParams(
            needs_layout_passes=False, use_tc_tiling_on_sc=False),
        scratch_types=[
            pltpu.VMEM((spt * m,), jnp.int32),
            pltpu.VMEM((2, G * m, m * k), jnp.float32),
            pltpu.SMEM((4, offs_np.shape[1]), jnp.int32),
            pltpu.VMEM((spt,), jnp.float32),
            pltpu.SemaphoreType.DMA,
            pltpu.SemaphoreType.DMA,
        ],
    )
    out = fn(xflat, tab)
    if bpad != b:
        out = out[:b]
    return out


# speedup vs baseline: 1.1294x; 1.1294x over previous
import functools

import jax
import jax.numpy as jnp
import numpy as np
from jax import lax
from jax.experimental import pallas as pl
from jax.experimental.pallas import tpu as pltpu
from jax.experimental.pallas import tpu_sc as plsc

NC = 2
NS = 16
NW = NC * NS
G = 4


def _ffm_body(m, k, spt, pairs, x_hbm, tab_hbm, out_hbm,
              xv, buf, accv, outv, sem0, sem1):
    ng = spt // G
    c = lax.axis_index("c")
    s = lax.axis_index("s")
    wid = s * NC + c
    base = wid * spt

    pltpu.sync_copy(x_hbm.at[pl.ds(base * m, spt * m)], xv)

    def issue(g, b):
        idx = xv.at[pl.ds(g * (G * m), G * m)]
        sem = sem0 if b == 0 else sem1
        pltpu.async_copy(tab_hbm.at[idx], buf.at[b], sem)

    def wait_grp(b):
        idx = xv.at[pl.ds(0, G * m)]
        sem = sem0 if b == 0 else sem1
        pltpu.make_async_copy(tab_hbm.at[idx], buf.at[b], sem).wait()

    issue(0, 0)

    lane = lax.iota(jnp.int32, 16)

    def sample_body(par, g, t, zvec):
        tm0 = t * m
        tm1 = tm0 + m
        zero = jnp.zeros((k,), jnp.float32)
        accs = (zero,) * 8

        def pair_prod(tm, i, j):
            v1 = buf[par, tm + i, pl.ds(j * k, k)]
            v2 = buf[par, tm + j, pl.ds(i * k, k)]
            return v1 * v2

        for d in range(1, m // 2):
            def diag_body(i, acc, d=d):
                jj = i + d
                j = jnp.where(jj >= m, jj - m, jj)
                return acc[2:] + (acc[0] + pair_prod(tm0, i, j),
                                  acc[1] + pair_prod(tm1, i, j))
            accs = plsc.parallel_loop(0, m, 1, unroll=8, carry=accs)(diag_body)

        half0, half1 = zero, zero
        for i in range(m // 2):
            half0 = half0 + pair_prod(tm0, i, i + m // 2)
            half1 = half1 + pair_prod(tm1, i, i + m // 2)
        tot0 = ((accs[0] + accs[2]) + (accs[4] + accs[6])) + half0
        tot1 = ((accs[1] + accs[3]) + (accs[5] + accs[7])) + half1
        z0 = lax.reduce_sum(tot0, axes=(0,))
        z1 = lax.reduce_sum(tot1, axes=(0,))
        ls = g * G + t
        zvec = jnp.where(lane == lax.rem(ls, 16), z0, zvec)
        zvec = jnp.where(lane == lax.rem(ls + 1, 16), z1, zvec)

        @pl.when(lax.rem(ls, 16) == 14)
        def _():
            chunk = (ls // 16) * 16
            outv[pl.ds(chunk, 16)] = 1.0 / (1.0 + jnp.exp(-zvec))

        return zvec

    def group_body(g, zvec):
        par = lax.rem(g, 2)
        more = g + 1 < ng

        @pl.when(jnp.logical_and(more, par == 0))
        def _():
            issue(g + 1, 1)

        @pl.when(jnp.logical_and(more, par == 1))
        def _():
            issue(g + 1, 0)

        @pl.when(par == 0)
        def _():
            wait_grp(0)

        @pl.when(par == 1)
        def _():
            wait_grp(1)

        return lax.fori_loop(
            0, G // 2, lambda h, zv: sample_body(par, g, h * 2, zv), zvec)

    lax.fori_loop(0, ng, group_body, jnp.zeros((16,), jnp.float32))

    pltpu.sync_copy(outv, out_hbm.at[pl.ds(base, spt)])


def kernel(x, field_embeddings):
    n, m, k = field_embeddings.shape
    if x.ndim == 1:
        x = x[None, :]
    b = x.shape[0]
    x = (x.astype(jnp.int32) % n).astype(jnp.int32)

    bpad = ((b + NW * 16 - 1) // (NW * 16)) * (NW * 16)
    if bpad != b:
        x = jnp.concatenate(
            [x, jnp.zeros((bpad - b, m), jnp.int32)], axis=0)
    spt = bpad // NW

    tab = field_embeddings.reshape(n, m * k)
    xflat = x.reshape(-1)
    pairs = [(int(i), int(j)) for i, j in zip(*np.triu_indices(m, k=1))]

    mesh = plsc.VectorSubcoreMesh(
        core_axis_name="c", subcore_axis_name="s",
        num_cores=NC, num_subcores=NS)
    fn = pl.kernel(
        functools.partial(_ffm_body, m, k, spt, pairs),
        out_type=jax.ShapeDtypeStruct((bpad,), jnp.float32),
        mesh=mesh,
        compiler_params=pltpu.CompilerParams(
            needs_layout_passes=False, use_tc_tiling_on_sc=False),
        scratch_types=[
            pltpu.VMEM((spt * m,), jnp.int32),
            pltpu.VMEM((2, G * m, m * k), jnp.float32),
            pltpu.VMEM((8, 16), jnp.float32),
            pltpu.VMEM((spt,), jnp.float32),
            pltpu.SemaphoreType.DMA,
            pltpu.SemaphoreType.DMA,
        ],
    )
    out = fn(xflat, tab)
    if bpad != b:
        out = out[:b]
    return out


# speedup vs baseline: 1.1342x; 1.0043x over previous
import functools

import jax
import jax.numpy as jnp
import numpy as np
from jax import lax
from jax.experimental import pallas as pl
from jax.experimental.pallas import tpu as pltpu
from jax.experimental.pallas import tpu_sc as plsc

NC = 2
NS = 16
NW = NC * NS
G = 4


def _ffm_body(m, k, spt, pairs, x_hbm, tab_hbm, out_hbm,
              xv, buf, accv, outv, sem0, sem1):
    ng = spt // G
    c = lax.axis_index("c")
    s = lax.axis_index("s")
    wid = s * NC + c
    base = wid * spt

    pltpu.sync_copy(x_hbm.at[pl.ds(base * m, spt * m)], xv)

    def issue(g, b):
        idx = xv.at[pl.ds(g * (G * m), G * m)]
        sem = sem0 if b == 0 else sem1
        pltpu.async_copy(tab_hbm.at[idx], buf.at[b], sem)

    def wait_grp(b):
        idx = xv.at[pl.ds(0, G * m)]
        sem = sem0 if b == 0 else sem1
        pltpu.make_async_copy(tab_hbm.at[idx], buf.at[b], sem).wait()

    issue(0, 0)

    lane = lax.iota(jnp.int32, 16)

    def sample_body(par, g, t, zvec):
        tm0 = t * m
        tm1 = tm0 + m
        tm2 = tm0 + 2 * m
        tm3 = tm0 + 3 * m
        zero = jnp.zeros((k,), jnp.float32)
        accs = (zero,) * 8

        def pair_prod(tm, i, j):
            v1 = buf[par, tm + i, pl.ds(j * k, k)]
            v2 = buf[par, tm + j, pl.ds(i * k, k)]
            return v1 * v2

        for d in range(1, m // 2):
            def diag_body(i, acc, d=d):
                jj = i + d
                j = jnp.where(jj >= m, jj - m, jj)
                return acc[4:] + (acc[0] + pair_prod(tm0, i, j),
                                  acc[1] + pair_prod(tm1, i, j),
                                  acc[2] + pair_prod(tm2, i, j),
                                  acc[3] + pair_prod(tm3, i, j))
            accs = plsc.parallel_loop(0, m, 1, unroll=4, carry=accs)(diag_body)

        half = [zero, zero, zero, zero]
        for i in range(m // 2):
            for q, tm in enumerate((tm0, tm1, tm2, tm3)):
                half[q] = half[q] + pair_prod(tm, i, i + m // 2)
        zs = []
        for q in range(4):
            tot = (accs[q] + accs[q + 4]) + half[q]
            zs.append(lax.reduce_sum(tot, axes=(0,)))
        ls = g * G + t
        for q in range(4):
            zvec = jnp.where(lane == lax.rem(ls + q, 16), zs[q], zvec)

        @pl.when(lax.rem(ls, 16) == 12)
        def _():
            chunk = (ls // 16) * 16
            outv[pl.ds(chunk, 16)] = 1.0 / (1.0 + jnp.exp(-zvec))

        return zvec

    def group_body(g, zvec):
        par = lax.rem(g, 2)
        more = g + 1 < ng

        @pl.when(jnp.logical_and(more, par == 0))
        def _():
            issue(g + 1, 1)

        @pl.when(jnp.logical_and(more, par == 1))
        def _():
            issue(g + 1, 0)

        @pl.when(par == 0)
        def _():
            wait_grp(0)

        @pl.when(par == 1)
        def _():
            wait_grp(1)

        return sample_body(par, g, 0, zvec)

    lax.fori_loop(0, ng, group_body, jnp.zeros((16,), jnp.float32))

    pltpu.sync_copy(outv, out_hbm.at[pl.ds(base, spt)])


def kernel(x, field_embeddings):
    n, m, k = field_embeddings.shape
    if x.ndim == 1:
        x = x[None, :]
    b = x.shape[0]
    x = (x.astype(jnp.int32) % n).astype(jnp.int32)

    bpad = ((b + NW * 16 - 1) // (NW * 16)) * (NW * 16)
    if bpad != b:
        x = jnp.concatenate(
            [x, jnp.zeros((bpad - b, m), jnp.int32)], axis=0)
    spt = bpad // NW

    tab = field_embeddings.reshape(n, m * k)
    xflat = x.reshape(-1)
    pairs = [(int(i), int(j)) for i, j in zip(*np.triu_indices(m, k=1))]

    mesh = plsc.VectorSubcoreMesh(
        core_axis_name="c", subcore_axis_name="s",
        num_cores=NC, num_subcores=NS)
    fn = pl.kernel(
        functools.partial(_ffm_body, m, k, spt, pairs),
        out_type=jax.ShapeDtypeStruct((bpad,), jnp.float32),
        mesh=mesh,
        compiler_params=pltpu.CompilerParams(
            needs_layout_passes=False, use_tc_tiling_on_sc=False),
        scratch_types=[
            pltpu.VMEM((spt * m,), jnp.int32),
            pltpu.VMEM((2, G * m, m * k), jnp.float32),
            pltpu.VMEM((8, 16), jnp.float32),
            pltpu.VMEM((spt,), jnp.float32),
            pltpu.SemaphoreType.DMA,
            pltpu.SemaphoreType.DMA,
        ],
    )
    out = fn(xflat, tab)
    if bpad != b:
        out = out[:b]
    return out
